# baseline (speedup 1.0000x reference)
import functools

import jax
import jax.numpy as jnp
from jax import lax
from jax.experimental import pallas as pl
from jax.experimental.pallas import tpu as pltpu
from jax.experimental.pallas import tpu_sc as plsc

_N_FIELDS = 26
_VOCAB = 100000
_EMB = 32
_N_CONT = 13
_BATCH = 16384
_H1, _H2 = 512, 256
_EPS = 1e-5

_ROWS = _BATCH * _N_FIELDS
_NW = 32
_ROWS_W = _ROWS // _NW
_CHUNK = 128
_NCHUNK = _ROWS_W // _CHUNK
_GROUP = 8
_NGROUP = _NCHUNK // _GROUP
_GROUP_ROWS = _GROUP * _CHUNK


_VT_FULL = _VOCAB // 128
_TAIL_W = _VOCAB - _VT_FULL * 128
_NTILE = _N_FIELDS * _VT_FULL
_CONV_NB = 6


def _sc_convert(table_t, tail_lines):
    mesh = plsc.VectorSubcoreMesh(core_axis_name="c", subcore_axis_name="s")
    n_lines = _N_FIELDS * _VOCAB * _EMB // 128

    @functools.partial(
        pl.kernel,
        out_type=jax.ShapeDtypeStruct((n_lines, 128), jnp.float32),
        mesh=mesh,
        scratch_types=[
            pltpu.VMEM((2 * _CONV_NB, _EMB, 128), jnp.float32),
            pltpu.VMEM((2 * _CONV_NB, _EMB, 128), jnp.float32),
            pltpu.SemaphoreType.DMA,
            pltpu.SemaphoreType.DMA,
        ],
        compiler_params=pltpu.CompilerParams(
            use_tc_tiling_on_sc=True, needs_layout_passes=False
        ),
    )
    def convert_kernel(in_hbm, tail_hbm, out_hbm, in_v, pack_v, sem_in, sem_out):
        wid = lax.axis_index("s") * 2 + lax.axis_index("c")
        start = (wid * _NTILE) // _NW
        end = ((wid + 1) * _NTILE) // _NW
        cnt = end - start

        def fv(t):
            f = t // _VT_FULL
            return f, t - f * _VT_FULL

        def step(f, vt):
            wrap = vt + 1 >= _VT_FULL
            return (jnp.where(wrap, f + 1, f).astype(jnp.int32),
                    jnp.where(wrap, 0, vt + 1).astype(jnp.int32))

        def in_copy(f, vt, p):
            return pltpu.make_async_copy(
                in_hbm.at[pl.ds(pl.multiple_of(f * _EMB, _EMB), _EMB),
                          pl.ds(pl.multiple_of(vt * 128, 128), 128)],
                in_v.at[p], sem_in,
            )

        def out_copy(f, vt, p):
            line0 = f * (_VOCAB * _EMB // 128) + vt * (128 * _EMB // 128)
            return pltpu.make_async_copy(
                pack_v.at[p],
                out_hbm.at[pl.ds(pl.multiple_of(line0, 8), _EMB)], sem_out,
            )

        def shuffle(p):
            def e_body(e, carry):
                for k in range(8):
                    c = lax.iota(jnp.int32, 16) + (16 * k)
                    row = c >> 2
                    col = ((c & 3) << 5) + e
                    v = in_v[p, e, pl.ds(16 * k, 16)]
                    plsc.store_scatter(pack_v.at[p], [row, col], v)
                return carry

            lax.fori_loop(0, _EMB, e_body, 0)

        nb = _CONV_NB
        ngrp = (_NTILE // _NW) // nb + 2

        def in_wait(p):
            in_copy(0, 0, p).wait()

        def out_wait(p):
            out_copy(0, 0, p).wait()

        f0, vt0 = fv(start)
        fa, vta = f0, vt0
        for b in range(nb):
            @pl.when(b < cnt)
            def _(fa=fa, vta=vta, b=b):
                in_copy(fa, vta, b).start()
            fa, vta = step(fa, vta)

        def group(G, carry):
            f, vt, fa, vta = carry
            s = (G & 1) * nb
            so = ((G + 1) & 1) * nb

            for b in range(nb):
                tf = (G + 1) * nb + b

                @pl.when(tf < cnt)
                def _(fa=fa, vta=vta, b=b):
                    in_copy(fa, vta, so + b).start()
                fa, vta = step(fa, vta)

            for b in range(nb):
                @pl.when(G * nb + b < cnt)
                def _(b=b):
                    in_wait(s + b)

            for b in range(nb):
                @pl.when(jnp.logical_and(G >= 2, (G - 2) * nb + b < cnt))
                def _(b=b):
                    out_wait(s + b)

            for b in range(nb):
                @pl.when(G * nb + b < cnt)
                def _(f=f, vt=vt, b=b):
                    shuffle(s + b)
                    out_copy(f, vt, s + b).start()
                f, vt = step(f, vt)

            return (f, vt, fa, vta)

        lax.fori_loop(0, ngrp, group, (f0, vt0, fa, vta))

        for b in range(2 * nb):
            t = (ngrp - 2) * nb + b

            @pl.when(t < cnt)
            def _(b=b):
                out_wait((((ngrp - 2) * nb + b) // nb % 2) * nb + b % nb)

        tl = _TAIL_W * _EMB // 128
        @pl.when(wid < _N_FIELDS)
        def _():
            f = wid
            pltpu.sync_copy(
                tail_hbm.at[pl.ds(pl.multiple_of(f * tl, 8), tl)],
                in_v.at[0, pl.ds(0, tl)],
            )
            line0 = (f * _VOCAB + _VT_FULL * 128) * _EMB // 128
            pltpu.sync_copy(
                in_v.at[0, pl.ds(0, tl)],
                out_hbm.at[pl.ds(pl.multiple_of(line0, 8), tl)],
            )

    return convert_kernel(table_t, tail_lines)


def _sc_gather(table128, idx3d):
    mesh = plsc.VectorSubcoreMesh(core_axis_name="c", subcore_axis_name="s")

    @functools.partial(
        pl.kernel,
        out_type=jax.ShapeDtypeStruct((_ROWS, _EMB), jnp.float32),
        mesh=mesh,
        scratch_types=[
            pltpu.VMEM((_NCHUNK, _CHUNK), jnp.int32),
            pltpu.VMEM((_GROUP_ROWS, _EMB), jnp.float32),
            pltpu.SemaphoreType.DMA,
        ],
        compiler_params=pltpu.CompilerParams(use_tc_tiling_on_sc=False),
    )
    def gather_kernel(table_hbm, idx_hbm, out_hbm, idx_v, rows_v, sem):
        table_rows = table_hbm
        out_rows = out_hbm
        wid = lax.axis_index("s") * 2 + lax.axis_index("c")
        base = wid * _ROWS_W
        pltpu.sync_copy(idx_hbm.at[wid], idx_v)

        def group_body(g, carry):
            copies = []
            for b in range(_GROUP):
                c = pltpu.async_copy(
                    table_rows.at[idx_v.at[g * _GROUP + b]],
                    rows_v.at[pl.ds(b * _CHUNK, _CHUNK)],
                    sem,
                )
                copies.append(c)
            for c in copies:
                c.wait()
            pltpu.sync_copy(
                rows_v, out_rows.at[pl.ds(base + g * _GROUP_ROWS, _GROUP_ROWS)]
            )
            return carry

        lax.fori_loop(0, _NGROUP, group_body, 0)

    return gather_kernel(table128, idx3d)


def _mlp(emb_flat, x_cont, w1e, w1c, b1f, w2f, b2f, w3f, b3f):
    bm = 512
    grid = (_BATCH // bm,)
    d_emb = _N_FIELDS * _EMB

    def body(emb_ref, cont_ref, w1e_ref, w1c_ref, b1_ref, w2_ref, b2_ref,
             w3_ref, b3_ref, out_ref):
        h = jnp.dot(emb_ref[...], w1e_ref[...], preferred_element_type=jnp.float32)
        h = h + jnp.dot(cont_ref[...], w1c_ref[...], preferred_element_type=jnp.float32)
        h = jnp.maximum(h + b1_ref[...], 0.0)
        h = jnp.dot(h, w2_ref[...], preferred_element_type=jnp.float32) + b2_ref[...]
        h = jnp.maximum(h, 0.0)
        out_ref[...] = (
            jnp.dot(h, w3_ref[...], preferred_element_type=jnp.float32) + b3_ref[...]
        )

    const = lambda shape: pl.BlockSpec(shape, lambda i: (0, 0))
    return pl.pallas_call(
        body,
        grid=grid,
        in_specs=[
            pl.BlockSpec((bm, d_emb), lambda i: (i, 0)),
            pl.BlockSpec((bm, _N_CONT), lambda i: (i, 0)),
            const((d_emb, _H1)),
            const((_N_CONT, _H1)),
            const((1, _H1)),
            const((_H1, _H2)),
            const((1, _H2)),
            const((_H2, 1)),
            const((1, 1)),
        ],
        out_specs=pl.BlockSpec((bm, 1), lambda i: (i, 0)),
        out_shape=jax.ShapeDtypeStruct((_BATCH, 1), jnp.float32),
    )(emb_flat, x_cont, w1e, w1c, b1f, w2f, b2f, w3f, b3f)


def kernel(x_cat, x_cont, tables, bn_c_g, bn_c_b, W1, b1, g1, be1, W2, b2,
           g2, be2, W3, b3):
    s = 1.0 / jnp.sqrt(1.0 + _EPS)
    d_emb = _N_FIELDS * _EMB

    xc = jnp.clip(x_cat, 0, _VOCAB - 1)
    flat_idx = (xc + jnp.arange(_N_FIELDS, dtype=jnp.int32)[None, :] * _VOCAB)
    idx3d = flat_idx.reshape(_NW, _NCHUNK, _CHUNK)
    table_t = tables.transpose(0, 2, 1).reshape(_N_FIELDS * _EMB, _VOCAB)
    tail_lines = tables[:, _VT_FULL * 128:, :].reshape(
        _N_FIELDS * _TAIL_W * _EMB // 128, 128)
    table2d = _sc_convert(table_t, tail_lines).reshape(_N_FIELDS * _VOCAB, _EMB)

    emb_flat = _sc_gather(table2d, idx3d).reshape(_BATCH, d_emb)

    w1e = W1[:d_emb]
    w1c_raw = W1[d_emb:]
    w1c = (bn_c_g * s)[:, None] * w1c_raw
    b1f = (b1 + bn_c_b @ w1c_raw)[None, :]
    w2f = (g1 * s)[:, None] * W2
    b2f = (b2 + be1 @ W2)[None, :]
    w3f = (g2 * s)[:, None] * W3
    b3f = (b3 + be2 @ W3)[None, :]

    return _mlp(emb_flat, x_cont, w1e, w1c, b1f, w2f, b2f, w3f, b3f)


# speedup vs baseline: 2.0730x; 2.0730x over previous
import functools

import jax
import jax.numpy as jnp
from jax import lax
from jax.experimental import pallas as pl
from jax.experimental.pallas import tpu as pltpu
from jax.experimental.pallas import tpu_sc as plsc

_N_FIELDS = 26
_VOCAB = 100000
_EMB = 32
_N_CONT = 13
_BATCH = 16384
_H1, _H2 = 512, 256
_EPS = 1e-5

_ROWS = _BATCH * _N_FIELDS
_NW = 32
_ROWS_W = _ROWS // _NW
_CHUNK = 128
_NCHUNK = _ROWS_W // _CHUNK
_GROUP = 8
_NGROUP = _NCHUNK // _GROUP
_GROUP_ROWS = _GROUP * _CHUNK


_VT_FULL = _VOCAB // 128
_TAIL_W = _VOCAB - _VT_FULL * 128
_NTILE = _N_FIELDS * _VT_FULL
_CONV_NB = 6


def _sc_convert(table_t, tail_lines):
    mesh = plsc.VectorSubcoreMesh(core_axis_name="c", subcore_axis_name="s")
    n_lines = _N_FIELDS * _VOCAB * _EMB // 128

    @functools.partial(
        pl.kernel,
        out_type=jax.ShapeDtypeStruct((n_lines, 128), jnp.float32),
        mesh=mesh,
        scratch_types=[
            pltpu.VMEM((2 * _CONV_NB, _EMB, 128), jnp.float32),
            pltpu.VMEM((2 * _CONV_NB, _EMB, 128), jnp.float32),
            pltpu.SemaphoreType.DMA,
            pltpu.SemaphoreType.DMA,
        ],
        compiler_params=pltpu.CompilerParams(
            use_tc_tiling_on_sc=True, needs_layout_passes=False
        ),
    )
    def convert_kernel(in_hbm, tail_hbm, out_hbm, in_v, pack_v, sem_in, sem_out):
        wid = lax.axis_index("s") * 2 + lax.axis_index("c")
        start = (wid * _NTILE) // _NW
        end = ((wid + 1) * _NTILE) // _NW
        cnt = end - start

        def fv(t):
            f = t // _VT_FULL
            return f, t - f * _VT_FULL

        def step(f, vt):
            wrap = vt + 1 >= _VT_FULL
            return (jnp.where(wrap, f + 1, f).astype(jnp.int32),
                    jnp.where(wrap, 0, vt + 1).astype(jnp.int32))

        def in_copy(f, vt, p):
            return pltpu.make_async_copy(
                in_hbm.at[pl.ds(pl.multiple_of(f * _EMB, _EMB), _EMB),
                          pl.ds(pl.multiple_of(vt * 128, 128), 128)],
                in_v.at[p], sem_in,
            )

        def out_copy(f, vt, p):
            line0 = f * (_VOCAB * _EMB // 128) + vt * (128 * _EMB // 128)
            return pltpu.make_async_copy(
                pack_v.at[p],
                out_hbm.at[pl.ds(pl.multiple_of(line0, 8), _EMB)], sem_out,
            )

        def shuffle(p):
            i16 = lax.iota(jnp.int32, 16)

            def m_body(m, carry):
                e0 = (m >> 3) << 4
                k = m & 7
                c = (k << 4) + i16
                for j in range(16):
                    e = e0 + ((j + i16) & 15)
                    v = plsc.load_gather(in_v.at[p], [e, c])
                    flat = (c << 5) + e
                    plsc.store_scatter(
                        pack_v.at[p], [flat >> 7, flat & 127], v)
                return carry

            lax.fori_loop(0, 16, m_body, 0)

        nb = _CONV_NB
        ngrp = (_NTILE // _NW) // nb + 2

        def in_wait(p):
            in_copy(0, 0, p).wait()

        def out_wait(p):
            out_copy(0, 0, p).wait()

        f0, vt0 = fv(start)
        fa, vta = f0, vt0
        for b in range(nb):
            @pl.when(b < cnt)
            def _(fa=fa, vta=vta, b=b):
                in_copy(fa, vta, b).start()
            fa, vta = step(fa, vta)

        def group(G, carry):
            f, vt, fa, vta = carry
            s = (G & 1) * nb
            so = ((G + 1) & 1) * nb

            for b in range(nb):
                tf = (G + 1) * nb + b

                @pl.when(tf < cnt)
                def _(fa=fa, vta=vta, b=b):
                    in_copy(fa, vta, so + b).start()
                fa, vta = step(fa, vta)

            for b in range(nb):
                @pl.when(G * nb + b < cnt)
                def _(b=b):
                    in_wait(s + b)

            for b in range(nb):
                @pl.when(jnp.logical_and(G >= 2, (G - 2) * nb + b < cnt))
                def _(b=b):
                    out_wait(s + b)

            for b in range(nb):
                @pl.when(G * nb + b < cnt)
                def _(f=f, vt=vt, b=b):
                    shuffle(s + b)
                    out_copy(f, vt, s + b).start()
                f, vt = step(f, vt)

            return (f, vt, fa, vta)

        lax.fori_loop(0, ngrp, group, (f0, vt0, fa, vta))

        for b in range(2 * nb):
            t = (ngrp - 2) * nb + b

            @pl.when(t < cnt)
            def _(b=b):
                out_wait((((ngrp - 2) * nb + b) // nb % 2) * nb + b % nb)

        tl = _TAIL_W * _EMB // 128
        @pl.when(wid < _N_FIELDS)
        def _():
            f = wid
            pltpu.sync_copy(
                tail_hbm.at[pl.ds(pl.multiple_of(f * tl, 8), tl)],
                in_v.at[0, pl.ds(0, tl)],
            )
            line0 = (f * _VOCAB + _VT_FULL * 128) * _EMB // 128
            pltpu.sync_copy(
                in_v.at[0, pl.ds(0, tl)],
                out_hbm.at[pl.ds(pl.multiple_of(line0, 8), tl)],
            )

    return convert_kernel(table_t, tail_lines)


def _sc_gather(table128, idx3d):
    mesh = plsc.VectorSubcoreMesh(core_axis_name="c", subcore_axis_name="s")

    @functools.partial(
        pl.kernel,
        out_type=jax.ShapeDtypeStruct((_ROWS, _EMB), jnp.float32),
        mesh=mesh,
        scratch_types=[
            pltpu.VMEM((_NCHUNK, _CHUNK), jnp.int32),
            pltpu.VMEM((_GROUP_ROWS, _EMB), jnp.float32),
            pltpu.SemaphoreType.DMA,
        ],
        compiler_params=pltpu.CompilerParams(use_tc_tiling_on_sc=False),
    )
    def gather_kernel(table_hbm, idx_hbm, out_hbm, idx_v, rows_v, sem):
        table_rows = table_hbm
        out_rows = out_hbm
        wid = lax.axis_index("s") * 2 + lax.axis_index("c")
        base = wid * _ROWS_W
        pltpu.sync_copy(idx_hbm.at[wid], idx_v)

        def group_body(g, carry):
            copies = []
            for b in range(_GROUP):
                c = pltpu.async_copy(
                    table_rows.at[idx_v.at[g * _GROUP + b]],
                    rows_v.at[pl.ds(b * _CHUNK, _CHUNK)],
                    sem,
                )
                copies.append(c)
            for c in copies:
                c.wait()
            pltpu.sync_copy(
                rows_v, out_rows.at[pl.ds(base + g * _GROUP_ROWS, _GROUP_ROWS)]
            )
            return carry

        lax.fori_loop(0, _NGROUP, group_body, 0)

    return gather_kernel(table128, idx3d)


def _mlp(emb_flat, x_cont, w1e, w1c, b1f, w2f, b2f, w3f, b3f):
    bm = 512
    grid = (_BATCH // bm,)
    d_emb = _N_FIELDS * _EMB

    def body(emb_ref, cont_ref, w1e_ref, w1c_ref, b1_ref, w2_ref, b2_ref,
             w3_ref, b3_ref, out_ref):
        h = jnp.dot(emb_ref[...], w1e_ref[...], preferred_element_type=jnp.float32)
        h = h + jnp.dot(cont_ref[...], w1c_ref[...], preferred_element_type=jnp.float32)
        h = jnp.maximum(h + b1_ref[...], 0.0)
        h = jnp.dot(h, w2_ref[...], preferred_element_type=jnp.float32) + b2_ref[...]
        h = jnp.maximum(h, 0.0)
        out_ref[...] = (
            jnp.dot(h, w3_ref[...], preferred_element_type=jnp.float32) + b3_ref[...]
        )

    const = lambda shape: pl.BlockSpec(shape, lambda i: (0, 0))
    return pl.pallas_call(
        body,
        grid=grid,
        in_specs=[
            pl.BlockSpec((bm, d_emb), lambda i: (i, 0)),
            pl.BlockSpec((bm, _N_CONT), lambda i: (i, 0)),
            const((d_emb, _H1)),
            const((_N_CONT, _H1)),
            const((1, _H1)),
            const((_H1, _H2)),
            const((1, _H2)),
            const((_H2, 1)),
            const((1, 1)),
        ],
        out_specs=pl.BlockSpec((bm, 1), lambda i: (i, 0)),
        out_shape=jax.ShapeDtypeStruct((_BATCH, 1), jnp.float32),
    )(emb_flat, x_cont, w1e, w1c, b1f, w2f, b2f, w3f, b3f)


def kernel(x_cat, x_cont, tables, bn_c_g, bn_c_b, W1, b1, g1, be1, W2, b2,
           g2, be2, W3, b3):
    s = 1.0 / jnp.sqrt(1.0 + _EPS)
    d_emb = _N_FIELDS * _EMB

    xc = jnp.clip(x_cat, 0, _VOCAB - 1)
    flat_idx = (xc + jnp.arange(_N_FIELDS, dtype=jnp.int32)[None, :] * _VOCAB)
    idx3d = flat_idx.reshape(_NW, _NCHUNK, _CHUNK)
    table_t = tables.transpose(0, 2, 1).reshape(_N_FIELDS * _EMB, _VOCAB)
    tail_lines = tables[:, _VT_FULL * 128:, :].reshape(
        _N_FIELDS * _TAIL_W * _EMB // 128, 128)
    table2d = _sc_convert(table_t, tail_lines).reshape(_N_FIELDS * _VOCAB, _EMB)

    emb_flat = _sc_gather(table2d, idx3d).reshape(_BATCH, d_emb)

    w1e = W1[:d_emb]
    w1c_raw = W1[d_emb:]
    w1c = (bn_c_g * s)[:, None] * w1c_raw
    b1f = (b1 + bn_c_b @ w1c_raw)[None, :]
    w2f = (g1 * s)[:, None] * W2
    b2f = (b2 + be1 @ W2)[None, :]
    w3f = (g2 * s)[:, None] * W3
    b3f = (b3 + be2 @ W3)[None, :]

    return _mlp(emb_flat, x_cont, w1e, w1c, b1f, w2f, b2f, w3f, b3f)


# speedup vs baseline: 2.0827x; 1.0047x over previous
import functools

import jax
import jax.numpy as jnp
from jax import lax
from jax.experimental import pallas as pl
from jax.experimental.pallas import tpu as pltpu
from jax.experimental.pallas import tpu_sc as plsc

_N_FIELDS = 26
_VOCAB = 100000
_EMB = 32
_N_CONT = 13
_BATCH = 16384
_H1, _H2 = 512, 256
_EPS = 1e-5

_ROWS = _BATCH * _N_FIELDS
_NW = 32
_ROWS_W = _ROWS // _NW
_CHUNK = 128
_NCHUNK = _ROWS_W // _CHUNK
_GROUP = 8
_NGROUP = _NCHUNK // _GROUP
_GROUP_ROWS = _GROUP * _CHUNK


_VT_FULL = _VOCAB // 128
_TAIL_W = _VOCAB - _VT_FULL * 128
_CONV_NB = 3
_CONV_C = 256
_VT2 = _VOCAB // _CONV_C
_MID_C0 = _VT2 * _CONV_C
_NTILE = _N_FIELDS * _VT2


def _sc_convert(table_t, tail_lines):
    mesh = plsc.VectorSubcoreMesh(core_axis_name="c", subcore_axis_name="s")
    n_lines = _N_FIELDS * _VOCAB * _EMB // 128

    @functools.partial(
        pl.kernel,
        out_type=jax.ShapeDtypeStruct((n_lines, 128), jnp.float32),
        mesh=mesh,
        scratch_types=[
            pltpu.VMEM((2 * _CONV_NB, _EMB, _CONV_C), jnp.float32),
            pltpu.VMEM((2 * _CONV_NB, _EMB * _CONV_C // 128, 128), jnp.float32),
            pltpu.SemaphoreType.DMA,
            pltpu.SemaphoreType.DMA,
        ],
        compiler_params=pltpu.CompilerParams(
            use_tc_tiling_on_sc=True, needs_layout_passes=False
        ),
    )
    def convert_kernel(in_hbm, tail_hbm, out_hbm, in_v, pack_v, sem_in, sem_out):
        wid = lax.axis_index("s") * 2 + lax.axis_index("c")
        start = (wid * _NTILE) // _NW
        end = ((wid + 1) * _NTILE) // _NW
        cnt = end - start

        def fv(t):
            f = t // _VT2
            return f, t - f * _VT2

        def step(f, vt):
            wrap = vt + 1 >= _VT2
            return (jnp.where(wrap, f + 1, f).astype(jnp.int32),
                    jnp.where(wrap, 0, vt + 1).astype(jnp.int32))

        def in_copy(f, vt, p):
            return pltpu.make_async_copy(
                in_hbm.at[pl.ds(pl.multiple_of(f * _EMB, _EMB), _EMB),
                          pl.ds(pl.multiple_of(vt * _CONV_C, _CONV_C), _CONV_C)],
                in_v.at[p], sem_in,
            )

        _OL = _CONV_C * _EMB // 128

        def out_copy(f, vt, p):
            line0 = f * (_VOCAB * _EMB // 128) + vt * _OL
            return pltpu.make_async_copy(
                pack_v.at[p],
                out_hbm.at[pl.ds(pl.multiple_of(line0, 8), _OL)], sem_out,
            )

        def shuffle(p):
            i16 = lax.iota(jnp.int32, 16)

            nk = _CONV_C // 16

            def m_body(m, carry):
                e0 = (m >= nk).astype(jnp.int32) << 4
                k = m - ((m >= nk).astype(jnp.int32) * nk)
                c = (k << 4) + i16
                for j in range(16):
                    e = e0 + ((j + i16) & 15)
                    v = plsc.load_gather(in_v.at[p], [e, c])
                    flat = (c << 5) + e
                    plsc.store_scatter(
                        pack_v.at[p], [flat >> 7, flat & 127], v)
                return carry

            lax.fori_loop(0, 2 * nk, m_body, 0)

        nb = _CONV_NB
        ngrp = (_NTILE // _NW) // nb + 2

        def in_wait(p):
            in_copy(0, 0, p).wait()

        def out_wait(p):
            out_copy(0, 0, p).wait()

        f0, vt0 = fv(start)
        fa, vta = f0, vt0
        for b in range(nb):
            @pl.when(b < cnt)
            def _(fa=fa, vta=vta, b=b):
                in_copy(fa, vta, b).start()
            fa, vta = step(fa, vta)

        def group(G, carry):
            f, vt, fa, vta = carry
            s = (G & 1) * nb
            so = ((G + 1) & 1) * nb

            for b in range(nb):
                tf = (G + 1) * nb + b

                @pl.when(tf < cnt)
                def _(fa=fa, vta=vta, b=b):
                    in_copy(fa, vta, so + b).start()
                fa, vta = step(fa, vta)

            for b in range(nb):
                @pl.when(G * nb + b < cnt)
                def _(b=b):
                    in_wait(s + b)

            for b in range(nb):
                @pl.when(jnp.logical_and(G >= 2, (G - 2) * nb + b < cnt))
                def _(b=b):
                    out_wait(s + b)

            for b in range(nb):
                @pl.when(G * nb + b < cnt)
                def _(f=f, vt=vt, b=b):
                    shuffle(s + b)
                    out_copy(f, vt, s + b).start()
                f, vt = step(f, vt)

            return (f, vt, fa, vta)

        lax.fori_loop(0, ngrp, group, (f0, vt0, fa, vta))

        for b in range(2 * nb):
            t = (ngrp - 2) * nb + b

            @pl.when(t < cnt)
            def _(b=b):
                out_wait((((ngrp - 2) * nb + b) // nb % 2) * nb + b % nb)

        i16b = lax.iota(jnp.int32, 16)

        @pl.when(wid < _N_FIELDS)
        def _():
            f = wid
            pltpu.sync_copy(
                in_hbm.at[pl.ds(pl.multiple_of(f * _EMB, _EMB), _EMB),
                          pl.ds(_MID_C0, 128)],
                in_v.at[0, :, pl.ds(0, 128)],
            )

            def m_body(m, carry):
                e0 = (m >= 8).astype(jnp.int32) << 4
                k = m - ((m >= 8).astype(jnp.int32) * 8)
                c = (k << 4) + i16b
                for j in range(16):
                    e = e0 + ((j + i16b) & 15)
                    v = plsc.load_gather(in_v.at[0], [e, c])
                    flat = (c << 5) + e
                    plsc.store_scatter(
                        pack_v.at[0], [flat >> 7, flat & 127], v)
                return carry

            lax.fori_loop(0, 16, m_body, 0)
            mid0 = f * (_VOCAB * _EMB // 128) + _MID_C0 * _EMB // 128
            pltpu.sync_copy(
                pack_v.at[0, pl.ds(0, 128 * _EMB // 128)],
                out_hbm.at[pl.ds(pl.multiple_of(mid0, 8), 128 * _EMB // 128)],
            )

        tl = _TAIL_W * _EMB // 128
        @pl.when(wid < _N_FIELDS)
        def _():
            f = wid
            pltpu.sync_copy(
                tail_hbm.at[pl.ds(pl.multiple_of(f * tl, 8), tl)],
                pack_v.at[0, pl.ds(0, tl)],
            )
            line0 = (f * _VOCAB + _VT_FULL * 128) * _EMB // 128
            pltpu.sync_copy(
                pack_v.at[0, pl.ds(0, tl)],
                out_hbm.at[pl.ds(pl.multiple_of(line0, 8), tl)],
            )

    return convert_kernel(table_t, tail_lines)


def _sc_gather(table128, idx3d):
    mesh = plsc.VectorSubcoreMesh(core_axis_name="c", subcore_axis_name="s")

    @functools.partial(
        pl.kernel,
        out_type=jax.ShapeDtypeStruct((_ROWS, _EMB), jnp.float32),
        mesh=mesh,
        scratch_types=[
            pltpu.VMEM((_NCHUNK, _CHUNK), jnp.int32),
            pltpu.VMEM((_GROUP_ROWS, _EMB), jnp.float32),
            pltpu.SemaphoreType.DMA,
        ],
        compiler_params=pltpu.CompilerParams(use_tc_tiling_on_sc=False),
    )
    def gather_kernel(table_hbm, idx_hbm, out_hbm, idx_v, rows_v, sem):
        table_rows = table_hbm
        out_rows = out_hbm
        wid = lax.axis_index("s") * 2 + lax.axis_index("c")
        base = wid * _ROWS_W
        pltpu.sync_copy(idx_hbm.at[wid], idx_v)

        def group_body(g, carry):
            copies = []
            for b in range(_GROUP):
                c = pltpu.async_copy(
                    table_rows.at[idx_v.at[g * _GROUP + b]],
                    rows_v.at[pl.ds(b * _CHUNK, _CHUNK)],
                    sem,
                )
                copies.append(c)
            for c in copies:
                c.wait()
            pltpu.sync_copy(
                rows_v, out_rows.at[pl.ds(base + g * _GROUP_ROWS, _GROUP_ROWS)]
            )
            return carry

        lax.fori_loop(0, _NGROUP, group_body, 0)

    return gather_kernel(table128, idx3d)


def _mlp(emb_flat, x_cont, w1e, w1c, b1f, w2f, b2f, w3f, b3f):
    bm = 512
    grid = (_BATCH // bm,)
    d_emb = _N_FIELDS * _EMB

    def body(emb_ref, cont_ref, w1e_ref, w1c_ref, b1_ref, w2_ref, b2_ref,
             w3_ref, b3_ref, out_ref):
        h = jnp.dot(emb_ref[...], w1e_ref[...], preferred_element_type=jnp.float32)
        h = h + jnp.dot(cont_ref[...], w1c_ref[...], preferred_element_type=jnp.float32)
        h = jnp.maximum(h + b1_ref[...], 0.0)
        h = jnp.dot(h, w2_ref[...], preferred_element_type=jnp.float32) + b2_ref[...]
        h = jnp.maximum(h, 0.0)
        out_ref[...] = (
            jnp.dot(h, w3_ref[...], preferred_element_type=jnp.float32) + b3_ref[...]
        )

    const = lambda shape: pl.BlockSpec(shape, lambda i: (0, 0))
    return pl.pallas_call(
        body,
        grid=grid,
        in_specs=[
            pl.BlockSpec((bm, d_emb), lambda i: (i, 0)),
            pl.BlockSpec((bm, _N_CONT), lambda i: (i, 0)),
            const((d_emb, _H1)),
            const((_N_CONT, _H1)),
            const((1, _H1)),
            const((_H1, _H2)),
            const((1, _H2)),
            const((_H2, 1)),
            const((1, 1)),
        ],
        out_specs=pl.BlockSpec((bm, 1), lambda i: (i, 0)),
        out_shape=jax.ShapeDtypeStruct((_BATCH, 1), jnp.float32),
    )(emb_flat, x_cont, w1e, w1c, b1f, w2f, b2f, w3f, b3f)


def kernel(x_cat, x_cont, tables, bn_c_g, bn_c_b, W1, b1, g1, be1, W2, b2,
           g2, be2, W3, b3):
    s = 1.0 / jnp.sqrt(1.0 + _EPS)
    d_emb = _N_FIELDS * _EMB

    xc = jnp.clip(x_cat, 0, _VOCAB - 1)
    flat_idx = (xc + jnp.arange(_N_FIELDS, dtype=jnp.int32)[None, :] * _VOCAB)
    idx3d = flat_idx.reshape(_NW, _NCHUNK, _CHUNK)
    table_t = tables.transpose(0, 2, 1).reshape(_N_FIELDS * _EMB, _VOCAB)
    tail_lines = tables[:, _VT_FULL * 128:, :].reshape(
        _N_FIELDS * _TAIL_W * _EMB // 128, 128)
    table2d = _sc_convert(table_t, tail_lines).reshape(_N_FIELDS * _VOCAB, _EMB)

    emb_flat = _sc_gather(table2d, idx3d).reshape(_BATCH, d_emb)

    w1e = W1[:d_emb]
    w1c_raw = W1[d_emb:]
    w1c = (bn_c_g * s)[:, None] * w1c_raw
    b1f = (b1 + bn_c_b @ w1c_raw)[None, :]
    w2f = (g1 * s)[:, None] * W2
    b2f = (b2 + be1 @ W2)[None, :]
    w3f = (g2 * s)[:, None] * W3
    b3f = (b3 + be2 @ W3)[None, :]

    return _mlp(emb_flat, x_cont, w1e, w1c, b1f, w2f, b2f, w3f, b3f)


# speedup vs baseline: 2.1796x; 1.0465x over previous
import functools

import jax
import jax.numpy as jnp
from jax import lax
from jax.experimental import pallas as pl
from jax.experimental.pallas import tpu as pltpu
from jax.experimental.pallas import tpu_sc as plsc

_N_FIELDS = 26
_VOCAB = 100000
_EMB = 32
_N_CONT = 13
_BATCH = 16384
_H1, _H2 = 512, 256
_EPS = 1e-5

_ROWS = _BATCH * _N_FIELDS
_NW = 32
_ROWS_W = _ROWS // _NW
_CHUNK = 128
_NCHUNK = _ROWS_W // _CHUNK
_GROUP = 8
_NGROUP = _NCHUNK // _GROUP
_GROUP_ROWS = _GROUP * _CHUNK


_VT_FULL = _VOCAB // 128
_TAIL_W = _VOCAB - _VT_FULL * 128
_CONV_NB = 3
_CONV_C = 256
_VT2 = _VOCAB // _CONV_C
_MID_C0 = _VT2 * _CONV_C
_NTILE = _N_FIELDS * _VT2


def _sc_convert(table_t, tail_lines):
    mesh = plsc.VectorSubcoreMesh(core_axis_name="c", subcore_axis_name="s")
    n_lines = _N_FIELDS * _VOCAB * _EMB // 128

    @functools.partial(
        pl.kernel,
        out_type=jax.ShapeDtypeStruct((n_lines, 128), jnp.float32),
        mesh=mesh,
        scratch_types=[
            pltpu.VMEM((2 * _CONV_NB, _EMB, _CONV_C), jnp.float32),
            pltpu.VMEM((2 * _CONV_NB, _EMB * _CONV_C // 128, 128), jnp.float32),
            pltpu.SemaphoreType.DMA,
            pltpu.SemaphoreType.DMA,
        ],
        compiler_params=pltpu.CompilerParams(
            use_tc_tiling_on_sc=True, needs_layout_passes=False
        ),
    )
    def convert_kernel(in_hbm, tail_hbm, out_hbm, in_v, pack_v, sem_in, sem_out):
        wid = lax.axis_index("s") * 2 + lax.axis_index("c")
        start = (wid * _NTILE) // _NW
        end = ((wid + 1) * _NTILE) // _NW
        cnt = end - start

        def fv(t):
            f = t // _VT2
            return f, t - f * _VT2

        def step(f, vt):
            wrap = vt + 1 >= _VT2
            return (jnp.where(wrap, f + 1, f).astype(jnp.int32),
                    jnp.where(wrap, 0, vt + 1).astype(jnp.int32))

        def in_copy(f, vt, p):
            return pltpu.make_async_copy(
                in_hbm.at[pl.ds(pl.multiple_of(f * _EMB, _EMB), _EMB),
                          pl.ds(pl.multiple_of(vt * _CONV_C, _CONV_C), _CONV_C)],
                in_v.at[p], sem_in,
            )

        _OL = _CONV_C * _EMB // 128

        def out_copy(f, vt, p):
            line0 = f * (_VOCAB * _EMB // 128) + vt * _OL
            return pltpu.make_async_copy(
                pack_v.at[p],
                out_hbm.at[pl.ds(pl.multiple_of(line0, 8), _OL)], sem_out,
            )

        def shuffle(p):
            i16 = lax.iota(jnp.int32, 16)

            nk = _CONV_C // 16

            def m_body(m, carry):
                c = (m << 4) + i16
                c32 = c << 5
                for half in (0, 16):
                    for j in range(16):
                        e = half + ((j + i16) & 15)
                        v = plsc.load_gather(in_v.at[p], [e, c])
                        flat = c32 + e
                        plsc.store_scatter(
                            pack_v.at[p], [flat >> 7, flat & 127], v)
                return carry

            lax.fori_loop(0, nk, m_body, 0)

        nb = _CONV_NB
        ngrp = (_NTILE // _NW) // nb + 2

        def in_wait(p):
            in_copy(0, 0, p).wait()

        def out_wait(p):
            out_copy(0, 0, p).wait()

        f0, vt0 = fv(start)
        fa, vta = f0, vt0
        for b in range(nb):
            @pl.when(b < cnt)
            def _(fa=fa, vta=vta, b=b):
                in_copy(fa, vta, b).start()
            fa, vta = step(fa, vta)

        def group(G, carry):
            f, vt, fa, vta = carry
            s = (G & 1) * nb
            so = ((G + 1) & 1) * nb

            for b in range(nb):
                tf = (G + 1) * nb + b

                @pl.when(tf < cnt)
                def _(fa=fa, vta=vta, b=b):
                    in_copy(fa, vta, so + b).start()
                fa, vta = step(fa, vta)

            for b in range(nb):
                @pl.when(G * nb + b < cnt)
                def _(b=b):
                    in_wait(s + b)

            for b in range(nb):
                @pl.when(jnp.logical_and(G >= 2, (G - 2) * nb + b < cnt))
                def _(b=b):
                    out_wait(s + b)

            for b in range(nb):
                @pl.when(G * nb + b < cnt)
                def _(f=f, vt=vt, b=b):
                    shuffle(s + b)
                    out_copy(f, vt, s + b).start()
                f, vt = step(f, vt)

            return (f, vt, fa, vta)

        lax.fori_loop(0, ngrp, group, (f0, vt0, fa, vta))

        for b in range(2 * nb):
            t = (ngrp - 2) * nb + b

            @pl.when(t < cnt)
            def _(b=b):
                out_wait((((ngrp - 2) * nb + b) // nb % 2) * nb + b % nb)

        i16b = lax.iota(jnp.int32, 16)

        @pl.when(wid < _N_FIELDS)
        def _():
            f = wid
            pltpu.sync_copy(
                in_hbm.at[pl.ds(pl.multiple_of(f * _EMB, _EMB), _EMB),
                          pl.ds(_MID_C0, 128)],
                in_v.at[0, :, pl.ds(0, 128)],
            )

            def m_body(m, carry):
                e0 = (m >= 8).astype(jnp.int32) << 4
                k = m - ((m >= 8).astype(jnp.int32) * 8)
                c = (k << 4) + i16b
                for j in range(16):
                    e = e0 + ((j + i16b) & 15)
                    v = plsc.load_gather(in_v.at[0], [e, c])
                    flat = (c << 5) + e
                    plsc.store_scatter(
                        pack_v.at[0], [flat >> 7, flat & 127], v)
                return carry

            lax.fori_loop(0, 16, m_body, 0)
            mid0 = f * (_VOCAB * _EMB // 128) + _MID_C0 * _EMB // 128
            pltpu.sync_copy(
                pack_v.at[0, pl.ds(0, 128 * _EMB // 128)],
                out_hbm.at[pl.ds(pl.multiple_of(mid0, 8), 128 * _EMB // 128)],
            )

        tl = _TAIL_W * _EMB // 128
        @pl.when(wid < _N_FIELDS)
        def _():
            f = wid
            pltpu.sync_copy(
                tail_hbm.at[pl.ds(pl.multiple_of(f * tl, 8), tl)],
                pack_v.at[0, pl.ds(0, tl)],
            )
            line0 = (f * _VOCAB + _VT_FULL * 128) * _EMB // 128
            pltpu.sync_copy(
                pack_v.at[0, pl.ds(0, tl)],
                out_hbm.at[pl.ds(pl.multiple_of(line0, 8), tl)],
            )

    return convert_kernel(table_t, tail_lines)


def _sc_gather(table128, idx3d):
    mesh = plsc.VectorSubcoreMesh(core_axis_name="c", subcore_axis_name="s")

    @functools.partial(
        pl.kernel,
        out_type=jax.ShapeDtypeStruct((_ROWS, _EMB), jnp.float32),
        mesh=mesh,
        scratch_types=[
            pltpu.VMEM((_NCHUNK, _CHUNK), jnp.int32),
            pltpu.VMEM((_GROUP_ROWS, _EMB), jnp.float32),
            pltpu.SemaphoreType.DMA,
        ],
        compiler_params=pltpu.CompilerParams(use_tc_tiling_on_sc=False),
    )
    def gather_kernel(table_hbm, idx_hbm, out_hbm, idx_v, rows_v, sem):
        table_rows = table_hbm
        out_rows = out_hbm
        wid = lax.axis_index("s") * 2 + lax.axis_index("c")
        base = wid * _ROWS_W
        pltpu.sync_copy(idx_hbm.at[wid], idx_v)

        def group_body(g, carry):
            copies = []
            for b in range(_GROUP):
                c = pltpu.async_copy(
                    table_rows.at[idx_v.at[g * _GROUP + b]],
                    rows_v.at[pl.ds(b * _CHUNK, _CHUNK)],
                    sem,
                )
                copies.append(c)
            for c in copies:
                c.wait()
            pltpu.sync_copy(
                rows_v, out_rows.at[pl.ds(base + g * _GROUP_ROWS, _GROUP_ROWS)]
            )
            return carry

        lax.fori_loop(0, _NGROUP, group_body, 0)

    return gather_kernel(table128, idx3d)


def _mlp(emb_flat, x_cont, w1e, w1c, b1f, w2f, b2f, w3f, b3f):
    bm = 512
    grid = (_BATCH // bm,)
    d_emb = _N_FIELDS * _EMB

    def body(emb_ref, cont_ref, w1e_ref, w1c_ref, b1_ref, w2_ref, b2_ref,
             w3_ref, b3_ref, out_ref):
        h = jnp.dot(emb_ref[...], w1e_ref[...], preferred_element_type=jnp.float32)
        h = h + jnp.dot(cont_ref[...], w1c_ref[...], preferred_element_type=jnp.float32)
        h = jnp.maximum(h + b1_ref[...], 0.0)
        h = jnp.dot(h, w2_ref[...], preferred_element_type=jnp.float32) + b2_ref[...]
        h = jnp.maximum(h, 0.0)
        out_ref[...] = (
            jnp.dot(h, w3_ref[...], preferred_element_type=jnp.float32) + b3_ref[...]
        )

    const = lambda shape: pl.BlockSpec(shape, lambda i: (0, 0))
    return pl.pallas_call(
        body,
        grid=grid,
        in_specs=[
            pl.BlockSpec((bm, d_emb), lambda i: (i, 0)),
            pl.BlockSpec((bm, _N_CONT), lambda i: (i, 0)),
            const((d_emb, _H1)),
            const((_N_CONT, _H1)),
            const((1, _H1)),
            const((_H1, _H2)),
            const((1, _H2)),
            const((_H2, 1)),
            const((1, 1)),
        ],
        out_specs=pl.BlockSpec((bm, 1), lambda i: (i, 0)),
        out_shape=jax.ShapeDtypeStruct((_BATCH, 1), jnp.float32),
    )(emb_flat, x_cont, w1e, w1c, b1f, w2f, b2f, w3f, b3f)


def kernel(x_cat, x_cont, tables, bn_c_g, bn_c_b, W1, b1, g1, be1, W2, b2,
           g2, be2, W3, b3):
    s = 1.0 / jnp.sqrt(1.0 + _EPS)
    d_emb = _N_FIELDS * _EMB

    xc = jnp.clip(x_cat, 0, _VOCAB - 1)
    flat_idx = (xc + jnp.arange(_N_FIELDS, dtype=jnp.int32)[None, :] * _VOCAB)
    idx3d = flat_idx.reshape(_NW, _NCHUNK, _CHUNK)
    table_t = tables.transpose(0, 2, 1).reshape(_N_FIELDS * _EMB, _VOCAB)
    tail_lines = tables[:, _VT_FULL * 128:, :].reshape(
        _N_FIELDS * _TAIL_W * _EMB // 128, 128)
    table2d = _sc_convert(table_t, tail_lines).reshape(_N_FIELDS * _VOCAB, _EMB)

    emb_flat = _sc_gather(table2d, idx3d).reshape(_BATCH, d_emb)

    w1e = W1[:d_emb]
    w1c_raw = W1[d_emb:]
    w1c = (bn_c_g * s)[:, None] * w1c_raw
    b1f = (b1 + bn_c_b @ w1c_raw)[None, :]
    w2f = (g1 * s)[:, None] * W2
    b2f = (b2 + be1 @ W2)[None, :]
    w3f = (g2 * s)[:, None] * W3
    b3f = (b3 + be2 @ W3)[None, :]

    return _mlp(emb_flat, x_cont, w1e, w1c, b1f, w2f, b2f, w3f, b3f)
